# Initial kernel scaffold; baseline (speedup 1.0000x reference)
import jax
import jax.numpy as jnp
from jax import lax
from jax.experimental import pallas as pl
from jax.experimental.pallas import tpu as pltpu
from jax.experimental.pallas import tpu_sc as plsc

D = 64
LANES = 16
VPT = D // LANES
BOM_ID = 3
NC, NS = 2, 16
NW = NC * NS
CHUNK = 512
SUB = 128
NSUB = CHUNK // SUB

G_TABLE = (0, 1, 1, 2, 2, 2, 2, 2, 3, 4, 5, 6, 4, 4)
NGATHER = len(G_TABLE)
Z_ROW = (16, 100000, 1000, 100000, 100000, 1000, 1000)


def _body(*refs):
    idx_hbm = refs[0:NGATHER]
    tabs = refs[NGATHER:NGATHER + 7]
    out = refs[NGATHER + 7]
    sc = NGATHER + 8
    idxb = refs[sc:sc + NGATHER]
    rowb = refs[sc + NGATHER]
    acc = refs[sc + NGATHER + 1]
    sem = refs[sc + NGATHER + 2]

    wid = lax.axis_index("s") * NC + lax.axis_index("c")
    rows_per_worker = 25600 // SUB
    base_row = wid * rows_per_worker

    def chunk(c, _):
        r0 = base_row + c * NSUB
        for g in range(NGATHER):
            pltpu.sync_copy(idx_hbm[g].at[pl.ds(r0, NSUB)], idxb[g])
        for s in range(NSUB):
            for i in range(SUB // LANES):
                sl = (s, pl.ds(i * LANES, LANES))
                tv = idxb[0][sl]
                m = tv == BOM_ID
                for g in range(1, NGATHER):
                    zk = jnp.full((LANES,), Z_ROW[G_TABLE[g]], jnp.int32)
                    iv = idxb[g][sl]
                    if g >= 12:
                        idxb[g][sl] = jnp.where(m, iv, zk)
                    else:
                        idxb[g][sl] = jnp.where(m, zk, iv)
                idxb[0][sl] = jnp.where(
                    m, jnp.full((LANES,), Z_ROW[0], jnp.int32), tv)
        for g in range(NGATHER):
            w_ref = tabs[G_TABLE[g]]
            cps = [
                pltpu.async_copy(
                    w_ref.at[idxb[g].at[s]],
                    rowb.at[pl.ds(s * SUB, SUB), :],
                    sem,
                )
                for s in range(NSUB)
            ]
            for cp in cps:
                cp.wait()
            if g == 0:
                def tok0(t, _):
                    for j in range(VPT):
                        jl = pl.ds(j * LANES, LANES)
                        acc[t, jl] = rowb[t, jl]
                    return ()
                lax.fori_loop(0, CHUNK, tok0, ())
            else:
                def tok(t, _):
                    for j in range(VPT):
                        jl = pl.ds(j * LANES, LANES)
                        plsc.addupdate(acc.at[t, jl], rowb[t, jl])
                    return ()
                lax.fori_loop(0, CHUNK, tok, ())
        pltpu.sync_copy(acc, out.at[pl.ds(r0 * SUB, CHUNK), :])
        return ()

    lax.fori_loop(0, 25600 // CHUNK, chunk, ())


def kernel(type, location, source_location, start_time, end_time,
           request_time, commit_time, lead_time, demand, material, method,
           quantity, parent, child, W_type, W_loc, W_time, W_demand, W_mat,
           W_method, W_qty):
    b, l = type.shape
    n = b * l
    idx_arrays = (type, location, source_location, start_time, end_time,
                  request_time, commit_time, lead_time, demand, material,
                  method, quantity, parent, child)
    idxs = [x.reshape(n // SUB, SUB) for x in idx_arrays]

    def zrow(w):
        return jnp.concatenate([w, jnp.zeros((1, D), w.dtype)], axis=0)

    tabs = [zrow(W_type), zrow(W_loc), zrow(W_time), zrow(W_demand),
            zrow(W_mat), zrow(W_method), zrow(W_qty)]

    mesh = plsc.VectorSubcoreMesh(core_axis_name="c", subcore_axis_name="s")
    scratch = ([pltpu.VMEM((NSUB, SUB), jnp.int32) for _ in range(NGATHER)]
               + [pltpu.VMEM((CHUNK, D), jnp.float32),
                  pltpu.VMEM((CHUNK, D), jnp.float32),
                  pltpu.SemaphoreType.DMA])
    out = pl.kernel(
        _body,
        out_type=jax.ShapeDtypeStruct((n, D), jnp.float32),
        mesh=mesh,
        scratch_types=scratch,
    )(*idxs, *tabs)
    return out.reshape(b, l, D)


# speedup vs baseline: 1.1978x; 1.1978x over previous
import jax
import jax.numpy as jnp
from jax import lax
from jax.experimental import pallas as pl
from jax.experimental.pallas import tpu as pltpu
from jax.experimental.pallas import tpu_sc as plsc

D = 64
LANES = 16
VPT = D // LANES
BOM_ID = 3
NC, NS = 2, 16
NW = NC * NS
CHUNK = 512
SUB = 128
NSUB = CHUNK // SUB

G_TABLE = (0, 1, 1, 2, 2, 2, 2, 2, 3, 4, 5, 6, 4, 4)
NGATHER = len(G_TABLE)
Z_ROW = (16, 100000, 1000, 100000, 100000, 1000, 1000)


def _body(*refs):
    idx_hbm = refs[0:NGATHER]
    tabs = refs[NGATHER:NGATHER + 7]
    out = refs[NGATHER + 7]
    sc = NGATHER + 8
    idxb = refs[sc:sc + NGATHER]
    rowb = refs[sc + NGATHER]
    acc = refs[sc + NGATHER + 1]
    sem = refs[sc + NGATHER + 2]

    wid = lax.axis_index("s") * NC + lax.axis_index("c")
    rows_per_worker = 25600 // SUB
    base_row = wid * rows_per_worker

    def chunk(c, _):
        r0 = base_row + c * NSUB
        for g in range(NGATHER):
            pltpu.sync_copy(idx_hbm[g].at[pl.ds(r0, NSUB)], idxb[g])
        for s in range(NSUB):
            for i in range(SUB // LANES):
                sl = (s, pl.ds(i * LANES, LANES))
                tv = idxb[0][sl]
                m = tv == BOM_ID
                for g in range(1, NGATHER):
                    zk = jnp.full((LANES,), Z_ROW[G_TABLE[g]], jnp.int32)
                    iv = idxb[g][sl]
                    if g >= 12:
                        idxb[g][sl] = jnp.where(m, iv, zk)
                    else:
                        idxb[g][sl] = jnp.where(m, zk, iv)
                idxb[0][sl] = jnp.where(
                    m, jnp.full((LANES,), Z_ROW[0], jnp.int32), tv)
        for g in range(NGATHER):
            w_ref = tabs[G_TABLE[g]]
            cps = [
                pltpu.async_copy(
                    w_ref.at[idxb[g].at[s]],
                    rowb.at[pl.ds(s * SUB, SUB), :],
                    sem,
                )
                for s in range(NSUB)
            ]
            for cp in cps:
                cp.wait()
            if g == 0:
                def tok0(t, _):
                    for j in range(VPT):
                        jl = pl.ds(j * LANES, LANES)
                        acc[t, jl] = rowb[t, jl]
                    return ()
                lax.fori_loop(0, CHUNK, tok0, ())
            else:
                def tok(t, _):
                    for j in range(VPT):
                        jl = pl.ds(j * LANES, LANES)
                        plsc.addupdate(acc.at[t, jl], rowb[t, jl])
                    return ()
                lax.fori_loop(0, CHUNK, tok, ())
        pltpu.sync_copy(acc, out.at[pl.ds(r0 * SUB, CHUNK), :])
        return ()

    lax.fori_loop(0, 25600 // CHUNK, chunk, ())


def kernel(type, location, source_location, start_time, end_time,
           request_time, commit_time, lead_time, demand, material, method,
           quantity, parent, child, W_type, W_loc, W_time, W_demand, W_mat,
           W_method, W_qty):
    b, l = type.shape
    n = b * l
    idx_arrays = (type, location, source_location, start_time, end_time,
                  request_time, commit_time, lead_time, demand, material,
                  method, quantity, parent, child)
    idxs = [x.reshape(n // SUB, SUB) for x in idx_arrays]

    def zrow(w):
        return jnp.concatenate([w, jnp.zeros((1, D), w.dtype)], axis=0)

    tabs = [zrow(W_type), zrow(W_loc), zrow(W_time), zrow(W_demand),
            zrow(W_mat), zrow(W_method), zrow(W_qty)]

    mesh = plsc.VectorSubcoreMesh(core_axis_name="c", subcore_axis_name="s")
    scratch = ([pltpu.VMEM((NSUB, SUB), jnp.int32) for _ in range(NGATHER)]
               + [pltpu.VMEM((CHUNK, D), jnp.float32),
                  pltpu.VMEM((CHUNK, D), jnp.float32),
                  pltpu.SemaphoreType.DMA])
    out = pl.kernel(
        _body,
        out_type=jax.ShapeDtypeStruct((n, D), jnp.float32),
        mesh=mesh,
        scratch_types=scratch,
        compiler_params=pltpu.CompilerParams(use_tc_tiling_on_sc=False),
    )(*idxs, *tabs)
    return out.reshape(b, l, D)
